# baseline (speedup 1.0000x reference)
import jax
import jax.numpy as jnp
from jax import lax
from jax.experimental import pallas as pl
from jax.experimental.pallas import tpu as pltpu
from jax.experimental.pallas import tpu_sc as plsc

N = 10000
E = 320000
D = 128

NC = 2
NS = 16
FG = 8
ECHUNK = E // NC
W_E = 640
N_WINDOWS = ECHUNK // W_E
NPAD = N + 1


def _mm_h_body(x_ref, fcw_ref, fcb_ref, h_ref):
    h_ref[...] = jax.nn.relu(
        jnp.dot(x_ref[...], fcw_ref[...].T, preferred_element_type=jnp.float32)
        + fcb_ref[...]
    )


def _compute_h(x, fc_W, fc_b):
    bn = 1000
    return pl.pallas_call(
        _mm_h_body,
        grid=(N // bn,),
        out_shape=jax.ShapeDtypeStruct((N, D), jnp.float32),
        in_specs=[
            pl.BlockSpec((bn, D), lambda i: (i, 0)),
            pl.BlockSpec((D, D), lambda i: (0, 0)),
            pl.BlockSpec((D,), lambda i: (0,)),
        ],
        out_specs=pl.BlockSpec((bn, D), lambda i: (i, 0)),
    )(x, fc_W, fc_b)


def _segmax_body(h_hbm, src_hbm, trg_hbm, out_hbm,
                 trgA, trgB, srcA, srcB, gidxA, gidxB, valsA, valsB,
                 agg, semA, semB, semIA, semIB):
    c = lax.axis_index("c")
    s = lax.axis_index("s")
    chunk_base = c * ECHUNK

    zeros16 = jnp.zeros((16,), jnp.float32)

    def zero_col(j, _):
        agg[pl.ds(j * 16, 16)] = zeros16
        return _
    lax.fori_loop(0, FG * NPAD // 16 + 1, zero_col, 0)

    iota = lax.iota(jnp.int32, 16)
    lane8 = jnp.bitwise_and(iota, 7)
    pair_base = (iota >= 8).astype(jnp.int32)
    swap_pat = jnp.bitwise_xor(iota, 1)
    lo_mask = iota < 8
    hi_mask = iota >= 8

    def issue_idx(w, src_buf, trg_buf, semi):
        woff = chunk_base + w * W_E
        pltpu.async_copy(src_hbm.at[pl.ds(woff, W_E)],
                         src_buf.at[pl.ds(0, W_E)], semi)
        pltpu.async_copy(trg_hbm.at[pl.ds(woff, W_E)], trg_buf, semi)

    def wait_idx(w, src_buf, trg_buf, semi):
        woff = chunk_base + w * W_E
        pltpu.make_async_copy(src_hbm.at[pl.ds(woff, W_E)],
                              src_buf.at[pl.ds(0, W_E)], semi).wait()
        pltpu.make_async_copy(trg_hbm.at[pl.ds(woff, W_E)],
                              trg_buf, semi).wait()

    def fire_gather(trg_buf, gidx_buf, vals_buf, sem):
        def build(k, _):
            o = k * 16
            gidx_buf[pl.ds(o, 16)] = trg_buf[pl.ds(o, 16)] * NS + s
            return _
        lax.fori_loop(0, W_E // 16, build, 0)
        pltpu.async_copy(h_hbm.at[gidx_buf], vals_buf, sem)

    def rmw_pair(src_buf, vals_buf, g, v, mask):
        pat = pair_base + 2 * v
        sel = plsc.load_gather(src_buf, [g * 16 + pat], mask=mask)
        aidx = lane8 * NPAD + sel
        vals = plsc.load_gather(vals_buf, [g * 16 + pat, lane8], mask=mask)
        cur = plsc.load_gather(agg, [aidx], mask=mask)
        plsc.store_scatter(agg, [aidx], jnp.maximum(cur, vals), mask=mask)

    def detect(src_buf, g):
        src_v = src_buf[pl.ds(g * 16, 16)]
        _cnt, last = plsc.scan_count(src_v)
        return jnp.all(last)

    def rmw_window(src_buf, vals_buf):
        def group(g, carry):
            nodup = detect(src_buf, g)
            aidxs, valss, curs = [], [], []
            for v in range(8):
                pat = pair_base + 2 * v
                sel = plsc.load_gather(src_buf, [g * 16 + pat])
                aidxs.append(lane8 * NPAD + sel)
            for v in range(8):
                pat = pair_base + 2 * v
                valss.append(plsc.load_gather(vals_buf,
                                              [g * 16 + pat, lane8]))
            for v in range(8):
                curs.append(plsc.load_gather(agg, [aidxs[v]]))
            for v in range(8):
                plsc.store_scatter(agg, [aidxs[v]],
                                   jnp.maximum(curs[v], valss[v]))

            @pl.when(jnp.logical_not(nodup))
            def _fixup():
                for v in range(8):
                    rmw_pair(src_buf, vals_buf, g, v, lo_mask)
                for v in range(8):
                    rmw_pair(src_buf, vals_buf, g, v, hi_mask)
            return carry
        lax.fori_loop(0, W_E // 16, group, 0)

    srcs = (srcA, srcB)
    trgs = (trgA, trgB)
    gidxs = (gidxA, gidxB)
    valss = (valsA, valsB)
    sems = (semA, semB)
    semis = (semIA, semIB)

    issue_idx(0, srcA, trgA, semIA)
    wait_idx(0, srcA, trgA, semIA)
    fire_gather(trgA, gidxA, valsA, semA)
    issue_idx(1, srcB, trgB, semIB)

    def outer(i, _):
        for b in range(2):
            w = i * 2 + b
            nxt = 1 - b

            @pl.when(w + 1 < N_WINDOWS)
            def _fire():
                wait_idx(w + 1, srcs[nxt], trgs[nxt], semis[nxt])
                fire_gather(trgs[nxt], gidxs[nxt], valss[nxt], sems[nxt])
            pltpu.make_async_copy(h_hbm.at[gidxs[b]], valss[b], sems[b]).wait()
            rmw_window(srcs[b], valss[b])

            @pl.when(w + 2 < N_WINDOWS)
            def _issue():
                issue_idx(w + 2, srcs[b], trgs[b], semis[b])
        return _
    lax.fori_loop(0, N_WINDOWS // 2, outer, 0)

    pltpu.sync_copy(agg.at[pl.ds(0, FG * NPAD)], out_hbm.at[c, s])


def _segmax(h_r, src, trg):
    mesh = plsc.VectorSubcoreMesh(core_axis_name="c", subcore_axis_name="s")
    kfn = pl.kernel(
        _segmax_body,
        out_type=jax.ShapeDtypeStruct((NC, NS, FG * NPAD), jnp.float32),
        mesh=mesh,
        compiler_params=pltpu.CompilerParams(use_tc_tiling_on_sc=False,
                                             needs_layout_passes=False),
        scratch_types=[
            pltpu.VMEM((W_E,), jnp.int32),
            pltpu.VMEM((W_E,), jnp.int32),
            pltpu.VMEM((W_E + 16,), jnp.int32),
            pltpu.VMEM((W_E + 16,), jnp.int32),
            pltpu.VMEM((W_E,), jnp.int32),
            pltpu.VMEM((W_E,), jnp.int32),
            pltpu.VMEM((W_E, FG), jnp.float32),
            pltpu.VMEM((W_E, FG), jnp.float32),
            pltpu.VMEM((FG * NPAD + 16,), jnp.float32),
            pltpu.SemaphoreType.DMA,
            pltpu.SemaphoreType.DMA,
            pltpu.SemaphoreType.DMA,
            pltpu.SemaphoreType.DMA,
        ],
    )
    return kfn(h_r, src, trg)


def _final_body(x_ref, a_ref, w_ref, out_ref):
    m = jnp.maximum(a_ref[0], a_ref[1])
    out_ref[...] = (
        jnp.dot(x_ref[...], w_ref[:D, :], preferred_element_type=jnp.float32)
        + lax.dot_general(m, w_ref[D:, :], (((0,), (0,)), ((), ())),
                          preferred_element_type=jnp.float32)
    )


def _final(x, aggT, W):
    return pl.pallas_call(
        _final_body,
        out_shape=jax.ShapeDtypeStruct((N, D), jnp.float32),
    )(x, aggT, W)


@jax.jit
def _run(x, idx, fc_W, fc_b, W):
    h = _compute_h(x, fc_W, fc_b)
    h_r = h.reshape(N * NS, FG)
    aggT = _segmax(h_r, idx[0], idx[1])
    aggT = aggT.reshape(NC, NS, FG, NPAD)[..., :N].reshape(NC, D, N)
    return _final(x, aggT, W)


def kernel(input_matrix, adjacency_coo_matrix, fc_W, fc_b, W):
    return _run(input_matrix, adjacency_coo_matrix, fc_W, fc_b, W)


# speedup vs baseline: 1.1417x; 1.1417x over previous
import jax
import jax.numpy as jnp
from jax import lax
from jax.experimental import pallas as pl
from jax.experimental.pallas import tpu as pltpu
from jax.experimental.pallas import tpu_sc as plsc

N = 10000
E = 320000
D = 128

NC = 2
NS = 16
FG = 8
ECHUNK = E // NC
W_E = 1600
N_WINDOWS = ECHUNK // W_E
NPAD = N + 1


def _mm_h_body(x_ref, fcw_ref, fcb_ref, h_ref):
    h_ref[...] = jax.nn.relu(
        jnp.dot(x_ref[...], fcw_ref[...].T, preferred_element_type=jnp.float32)
        + fcb_ref[...]
    )


def _compute_h(x, fc_W, fc_b):
    bn = 1000
    return pl.pallas_call(
        _mm_h_body,
        grid=(N // bn,),
        out_shape=jax.ShapeDtypeStruct((N, D), jnp.float32),
        in_specs=[
            pl.BlockSpec((bn, D), lambda i: (i, 0)),
            pl.BlockSpec((D, D), lambda i: (0, 0)),
            pl.BlockSpec((D,), lambda i: (0,)),
        ],
        out_specs=pl.BlockSpec((bn, D), lambda i: (i, 0)),
    )(x, fc_W, fc_b)


def _segmax_body(h_hbm, src_hbm, trg_hbm, out_hbm,
                 trgA, trgB, srcA, srcB, gidxA, gidxB, valsA, valsB,
                 agg, semA, semB, semIA, semIB):
    c = lax.axis_index("c")
    s = lax.axis_index("s")
    chunk_base = c * ECHUNK

    zeros16 = jnp.zeros((16,), jnp.float32)

    def zero_col(j, _):
        agg[pl.ds(j * 16, 16)] = zeros16
        return _
    lax.fori_loop(0, FG * NPAD // 16 + 1, zero_col, 0)

    iota = lax.iota(jnp.int32, 16)
    lane8 = jnp.bitwise_and(iota, 7)
    pair_base = (iota >= 8).astype(jnp.int32)
    swap_pat = jnp.bitwise_xor(iota, 1)
    lo_mask = iota < 8
    hi_mask = iota >= 8

    def issue_idx(w, src_buf, trg_buf, semi):
        woff = chunk_base + w * W_E
        pltpu.async_copy(src_hbm.at[pl.ds(woff, W_E)],
                         src_buf.at[pl.ds(0, W_E)], semi)
        pltpu.async_copy(trg_hbm.at[pl.ds(woff, W_E)], trg_buf, semi)

    def wait_idx(w, src_buf, trg_buf, semi):
        woff = chunk_base + w * W_E
        pltpu.make_async_copy(src_hbm.at[pl.ds(woff, W_E)],
                              src_buf.at[pl.ds(0, W_E)], semi).wait()
        pltpu.make_async_copy(trg_hbm.at[pl.ds(woff, W_E)],
                              trg_buf, semi).wait()

    def fire_gather(trg_buf, gidx_buf, vals_buf, sem):
        def build(k, _):
            o = k * 16
            gidx_buf[pl.ds(o, 16)] = trg_buf[pl.ds(o, 16)] * NS + s
            return _
        lax.fori_loop(0, W_E // 16, build, 0)
        pltpu.async_copy(h_hbm.at[gidx_buf], vals_buf, sem)

    def rmw_pair(src_buf, vals_buf, g, v, mask):
        pat = pair_base + 2 * v
        sel = plsc.load_gather(src_buf, [g * 16 + pat], mask=mask)
        aidx = lane8 * NPAD + sel
        vals = plsc.load_gather(vals_buf, [g * 16 + pat, lane8], mask=mask)
        cur = plsc.load_gather(agg, [aidx], mask=mask)
        plsc.store_scatter(agg, [aidx], jnp.maximum(cur, vals), mask=mask)

    def detect(src_buf, g):
        src_v = src_buf[pl.ds(g * 16, 16)]
        _cnt, last = plsc.scan_count(src_v)
        return jnp.all(last)

    def rmw_window(src_buf, vals_buf):
        def group(g, carry):
            nodup = detect(src_buf, g)
            aidxs, valss, curs = [], [], []
            for v in range(8):
                pat = pair_base + 2 * v
                sel = plsc.load_gather(src_buf, [g * 16 + pat])
                aidxs.append(lane8 * NPAD + sel)
            for v in range(8):
                pat = pair_base + 2 * v
                valss.append(plsc.load_gather(vals_buf,
                                              [g * 16 + pat, lane8]))
            for v in range(8):
                curs.append(plsc.load_gather(agg, [aidxs[v]]))
            for v in range(8):
                plsc.store_scatter(agg, [aidxs[v]],
                                   jnp.maximum(curs[v], valss[v]))

            @pl.when(jnp.logical_not(nodup))
            def _fixup():
                for v in range(8):
                    rmw_pair(src_buf, vals_buf, g, v, lo_mask)
                for v in range(8):
                    rmw_pair(src_buf, vals_buf, g, v, hi_mask)
            return carry
        lax.fori_loop(0, W_E // 16, group, 0)

    srcs = (srcA, srcB)
    trgs = (trgA, trgB)
    gidxs = (gidxA, gidxB)
    valss = (valsA, valsB)
    sems = (semA, semB)
    semis = (semIA, semIB)

    issue_idx(0, srcA, trgA, semIA)
    wait_idx(0, srcA, trgA, semIA)
    fire_gather(trgA, gidxA, valsA, semA)
    issue_idx(1, srcB, trgB, semIB)

    def outer(i, _):
        for b in range(2):
            w = i * 2 + b
            nxt = 1 - b

            @pl.when(w + 1 < N_WINDOWS)
            def _fire():
                wait_idx(w + 1, srcs[nxt], trgs[nxt], semis[nxt])
                fire_gather(trgs[nxt], gidxs[nxt], valss[nxt], sems[nxt])
            pltpu.make_async_copy(h_hbm.at[gidxs[b]], valss[b], sems[b]).wait()
            rmw_window(srcs[b], valss[b])

            @pl.when(w + 2 < N_WINDOWS)
            def _issue():
                issue_idx(w + 2, srcs[b], trgs[b], semis[b])
        return _
    lax.fori_loop(0, N_WINDOWS // 2, outer, 0)

    pltpu.sync_copy(agg.at[pl.ds(0, FG * NPAD)], out_hbm.at[c, s])


def _segmax(h_r, src, trg):
    mesh = plsc.VectorSubcoreMesh(core_axis_name="c", subcore_axis_name="s")
    kfn = pl.kernel(
        _segmax_body,
        out_type=jax.ShapeDtypeStruct((NC, NS, FG * NPAD), jnp.float32),
        mesh=mesh,
        compiler_params=pltpu.CompilerParams(use_tc_tiling_on_sc=False,
                                             needs_layout_passes=False),
        scratch_types=[
            pltpu.VMEM((W_E,), jnp.int32),
            pltpu.VMEM((W_E,), jnp.int32),
            pltpu.VMEM((W_E + 16,), jnp.int32),
            pltpu.VMEM((W_E + 16,), jnp.int32),
            pltpu.VMEM((W_E,), jnp.int32),
            pltpu.VMEM((W_E,), jnp.int32),
            pltpu.VMEM((W_E, FG), jnp.float32),
            pltpu.VMEM((W_E, FG), jnp.float32),
            pltpu.VMEM((FG * NPAD + 16,), jnp.float32),
            pltpu.SemaphoreType.DMA,
            pltpu.SemaphoreType.DMA,
            pltpu.SemaphoreType.DMA,
            pltpu.SemaphoreType.DMA,
        ],
    )
    return kfn(h_r, src, trg)


def _final_body(x_ref, a_ref, w_ref, out_ref):
    m = jnp.maximum(a_ref[0], a_ref[1])
    out_ref[...] = (
        jnp.dot(x_ref[...], w_ref[:D, :], preferred_element_type=jnp.float32)
        + lax.dot_general(m, w_ref[D:, :], (((0,), (0,)), ((), ())),
                          preferred_element_type=jnp.float32)
    )


def _final(x, aggT, W):
    return pl.pallas_call(
        _final_body,
        out_shape=jax.ShapeDtypeStruct((N, D), jnp.float32),
    )(x, aggT, W)


@jax.jit
def _run(x, idx, fc_W, fc_b, W):
    h = _compute_h(x, fc_W, fc_b)
    h_r = h.reshape(N * NS, FG)
    aggT = _segmax(h_r, idx[0], idx[1])
    aggT = aggT.reshape(NC, NS, FG, NPAD)[..., :N].reshape(NC, D, N)
    return _final(x, aggT, W)


def kernel(input_matrix, adjacency_coo_matrix, fc_W, fc_b, W):
    return _run(input_matrix, adjacency_coo_matrix, fc_W, fc_b, W)


# speedup vs baseline: 1.2433x; 1.0890x over previous
import jax
import jax.numpy as jnp
from jax import lax
from jax.experimental import pallas as pl
from jax.experimental.pallas import tpu as pltpu
from jax.experimental.pallas import tpu_sc as plsc

N = 10000
E = 320000
D = 128

NC = 2
NS = 16
FG = 8
ECHUNK = E // NC
W_E = 1600
N_WINDOWS = ECHUNK // W_E
NPAD = N + 1


def _mm_h_body(x_ref, fcw_ref, fcb_ref, h_ref):
    h_ref[...] = jax.nn.relu(
        jnp.dot(x_ref[...], fcw_ref[...].T, preferred_element_type=jnp.float32)
        + fcb_ref[...]
    )


def _compute_h(x, fc_W, fc_b):
    bn = 1000
    return pl.pallas_call(
        _mm_h_body,
        grid=(N // bn,),
        out_shape=jax.ShapeDtypeStruct((N, D), jnp.float32),
        in_specs=[
            pl.BlockSpec((bn, D), lambda i: (i, 0)),
            pl.BlockSpec((D, D), lambda i: (0, 0)),
            pl.BlockSpec((D,), lambda i: (0,)),
        ],
        out_specs=pl.BlockSpec((bn, D), lambda i: (i, 0)),
    )(x, fc_W, fc_b)


def _segmax_body(h_hbm, src_hbm, trg_hbm, out_hbm,
                 trgA, trgB, srcA, srcB, gidxA, gidxB, valsA, valsB,
                 agg, semA, semB, semIA, semIB):
    c = lax.axis_index("c")
    s = lax.axis_index("s")
    chunk_base = c * ECHUNK

    zeros16 = jnp.zeros((16,), jnp.float32)

    def zero_col(j, _):
        agg[pl.ds(j * 16, 16)] = zeros16
        return _
    lax.fori_loop(0, FG * NPAD // 16 + 1, zero_col, 0)

    iota = lax.iota(jnp.int32, 16)
    lane8 = jnp.bitwise_and(iota, 7)
    pair_base = (iota >= 8).astype(jnp.int32)
    swap_pat = jnp.bitwise_xor(iota, 1)
    lo_mask = iota < 8
    hi_mask = iota >= 8

    def issue_idx(w, src_buf, trg_buf, semi):
        woff = chunk_base + w * W_E
        pltpu.async_copy(src_hbm.at[pl.ds(woff, W_E)],
                         src_buf.at[pl.ds(0, W_E)], semi)
        pltpu.async_copy(trg_hbm.at[pl.ds(woff, W_E)], trg_buf, semi)

    def wait_idx(w, src_buf, trg_buf, semi):
        woff = chunk_base + w * W_E
        pltpu.make_async_copy(src_hbm.at[pl.ds(woff, W_E)],
                              src_buf.at[pl.ds(0, W_E)], semi).wait()
        pltpu.make_async_copy(trg_hbm.at[pl.ds(woff, W_E)],
                              trg_buf, semi).wait()

    def fire_gather(trg_buf, gidx_buf, vals_buf, sem):
        def build(k, _):
            o = k * 16
            gidx_buf[pl.ds(o, 16)] = trg_buf[pl.ds(o, 16)] * NS + s
            return _
        lax.fori_loop(0, W_E // 16, build, 0)
        pltpu.async_copy(h_hbm.at[gidx_buf], vals_buf, sem)

    def rmw_pair(src_buf, vals_buf, g, v, mask):
        pat = pair_base + 2 * v
        sel = plsc.load_gather(src_buf, [g * 16 + pat], mask=mask)
        aidx = lane8 * NPAD + sel
        vals = plsc.load_gather(vals_buf, [g * 16 + pat, lane8], mask=mask)
        cur = plsc.load_gather(agg, [aidx], mask=mask)
        plsc.store_scatter(agg, [aidx], jnp.maximum(cur, vals), mask=mask)

    def detect(src_buf, g):
        src_v = src_buf[pl.ds(g * 16, 16)]
        _cnt, last = plsc.scan_count(src_v)
        return jnp.all(last)

    def rmw_window(src_buf, vals_buf):
        def group2(t, carry):
            g0 = t * 2
            g1 = g0 + 1
            nodup = jnp.logical_and(detect(src_buf, g0), detect(src_buf, g1))
            aidxs, valss = [], []
            for g in (g0, g1):
                for v in range(8):
                    pat = pair_base + 2 * v
                    sel = plsc.load_gather(src_buf, [g * 16 + pat])
                    aidxs.append(lane8 * NPAD + sel)
            for g in (g0, g1):
                for v in range(8):
                    pat = pair_base + 2 * v
                    valss.append(plsc.load_gather(vals_buf,
                                                  [g * 16 + pat, lane8]))
            for half in range(2):
                curs = []
                for v in range(8):
                    curs.append(plsc.load_gather(agg, [aidxs[half * 8 + v]]))
                for v in range(8):
                    plsc.store_scatter(
                        agg, [aidxs[half * 8 + v]],
                        jnp.maximum(curs[v], valss[half * 8 + v]))

            @pl.when(jnp.logical_not(nodup))
            def _fixup():
                for g in (g0, g1):
                    for v in range(8):
                        rmw_pair(src_buf, vals_buf, g, v, lo_mask)
                    for v in range(8):
                        rmw_pair(src_buf, vals_buf, g, v, hi_mask)
            return carry
        lax.fori_loop(0, W_E // 32, group2, 0)

    srcs = (srcA, srcB)
    trgs = (trgA, trgB)
    gidxs = (gidxA, gidxB)
    valss = (valsA, valsB)
    sems = (semA, semB)
    semis = (semIA, semIB)

    issue_idx(0, srcA, trgA, semIA)
    wait_idx(0, srcA, trgA, semIA)
    fire_gather(trgA, gidxA, valsA, semA)
    issue_idx(1, srcB, trgB, semIB)

    def outer(i, _):
        for b in range(2):
            w = i * 2 + b
            nxt = 1 - b

            @pl.when(w + 1 < N_WINDOWS)
            def _fire():
                wait_idx(w + 1, srcs[nxt], trgs[nxt], semis[nxt])
                fire_gather(trgs[nxt], gidxs[nxt], valss[nxt], sems[nxt])
            pltpu.make_async_copy(h_hbm.at[gidxs[b]], valss[b], sems[b]).wait()
            rmw_window(srcs[b], valss[b])

            @pl.when(w + 2 < N_WINDOWS)
            def _issue():
                issue_idx(w + 2, srcs[b], trgs[b], semis[b])
        return _
    lax.fori_loop(0, N_WINDOWS // 2, outer, 0)

    pltpu.sync_copy(agg.at[pl.ds(0, FG * NPAD)], out_hbm.at[c, s])


def _segmax(h_r, src, trg):
    mesh = plsc.VectorSubcoreMesh(core_axis_name="c", subcore_axis_name="s")
    kfn = pl.kernel(
        _segmax_body,
        out_type=jax.ShapeDtypeStruct((NC, NS, FG * NPAD), jnp.float32),
        mesh=mesh,
        compiler_params=pltpu.CompilerParams(use_tc_tiling_on_sc=False,
                                             needs_layout_passes=False),
        scratch_types=[
            pltpu.VMEM((W_E,), jnp.int32),
            pltpu.VMEM((W_E,), jnp.int32),
            pltpu.VMEM((W_E + 16,), jnp.int32),
            pltpu.VMEM((W_E + 16,), jnp.int32),
            pltpu.VMEM((W_E,), jnp.int32),
            pltpu.VMEM((W_E,), jnp.int32),
            pltpu.VMEM((W_E, FG), jnp.float32),
            pltpu.VMEM((W_E, FG), jnp.float32),
            pltpu.VMEM((FG * NPAD + 16,), jnp.float32),
            pltpu.SemaphoreType.DMA,
            pltpu.SemaphoreType.DMA,
            pltpu.SemaphoreType.DMA,
            pltpu.SemaphoreType.DMA,
        ],
    )
    return kfn(h_r, src, trg)


def _final_body(x_ref, a_ref, w_ref, out_ref):
    m = jnp.maximum(a_ref[0], a_ref[1])
    out_ref[...] = (
        jnp.dot(x_ref[...], w_ref[:D, :], preferred_element_type=jnp.float32)
        + lax.dot_general(m, w_ref[D:, :], (((0,), (0,)), ((), ())),
                          preferred_element_type=jnp.float32)
    )


def _final(x, aggT, W):
    return pl.pallas_call(
        _final_body,
        out_shape=jax.ShapeDtypeStruct((N, D), jnp.float32),
    )(x, aggT, W)


@jax.jit
def _run(x, idx, fc_W, fc_b, W):
    h = _compute_h(x, fc_W, fc_b)
    h_r = h.reshape(N * NS, FG)
    aggT = _segmax(h_r, idx[0], idx[1])
    aggT = aggT.reshape(NC, NS, FG, NPAD)[..., :N].reshape(NC, D, N)
    return _final(x, aggT, W)


def kernel(input_matrix, adjacency_coo_matrix, fc_W, fc_b, W):
    return _run(input_matrix, adjacency_coo_matrix, fc_W, fc_b, W)


# speedup vs baseline: 1.4034x; 1.1288x over previous
import jax
import jax.numpy as jnp
from jax import lax
from jax.experimental import pallas as pl
from jax.experimental.pallas import tpu as pltpu
from jax.experimental.pallas import tpu_sc as plsc

N = 10000
E = 320000
D = 128

NC = 2
NS = 16
FG = 8
ECHUNK = E // NC
W_E = 1600
N_WINDOWS = ECHUNK // W_E
NPAD = N + 1


def _mm_h_body(x_ref, fcw_ref, fcb_ref, h_ref):
    h_ref[...] = jax.nn.relu(
        jnp.dot(x_ref[...], fcw_ref[...].T, preferred_element_type=jnp.float32)
        + fcb_ref[...]
    )


def _compute_h(x, fc_W, fc_b):
    bn = 1000
    return pl.pallas_call(
        _mm_h_body,
        grid=(N // bn,),
        out_shape=jax.ShapeDtypeStruct((N, D), jnp.float32),
        in_specs=[
            pl.BlockSpec((bn, D), lambda i: (i, 0)),
            pl.BlockSpec((D, D), lambda i: (0, 0)),
            pl.BlockSpec((D,), lambda i: (0,)),
        ],
        out_specs=pl.BlockSpec((bn, D), lambda i: (i, 0)),
    )(x, fc_W, fc_b)


def _segmax_body(h_hbm, src_hbm, trg_hbm, out_hbm,
                 trgA, trgB, srcA, srcB, gidxA, gidxB, valsA, valsB,
                 agg, semA, semB, semIA, semIB):
    c = lax.axis_index("c")
    s = lax.axis_index("s")
    chunk_base = c * ECHUNK

    zeros16 = jnp.zeros((16,), jnp.float32)

    def zero_col(j, _):
        agg[pl.ds(j * 16, 16)] = zeros16
        return _
    lax.fori_loop(0, FG * NPAD // 16 + 1, zero_col, 0)

    iota = lax.iota(jnp.int32, 16)
    lane8 = jnp.bitwise_and(iota, 7)
    pair_base = (iota >= 8).astype(jnp.int32)
    swap_pat = jnp.bitwise_xor(iota, 1)
    lo_mask = iota < 8
    hi_mask = iota >= 8

    def issue_idx(w, src_buf, trg_buf, semi):
        woff = chunk_base + w * W_E
        pltpu.async_copy(src_hbm.at[pl.ds(woff, W_E)],
                         src_buf.at[pl.ds(0, W_E)], semi)
        pltpu.async_copy(trg_hbm.at[pl.ds(woff, W_E)], trg_buf, semi)

    def wait_idx(w, src_buf, trg_buf, semi):
        woff = chunk_base + w * W_E
        pltpu.make_async_copy(src_hbm.at[pl.ds(woff, W_E)],
                              src_buf.at[pl.ds(0, W_E)], semi).wait()
        pltpu.make_async_copy(trg_hbm.at[pl.ds(woff, W_E)],
                              trg_buf, semi).wait()

    def fire_gather(trg_buf, gidx_buf, vals_buf, sem):
        def build(k, _):
            o = k * 16
            gidx_buf[pl.ds(o, 16)] = trg_buf[pl.ds(o, 16)] * NS + s
            return _
        lax.fori_loop(0, W_E // 16, build, 0)
        pltpu.async_copy(h_hbm.at[gidx_buf], vals_buf, sem)

    def rmw_pair(src_buf, vals_buf, g, v, mask):
        pat = pair_base + 2 * v
        sel = plsc.load_gather(src_buf, [g * 16 + pat], mask=mask)
        aidx = lane8 * NPAD + sel
        vals = plsc.load_gather(vals_buf, [g * 16 + pat, lane8], mask=mask)
        cur = plsc.load_gather(agg, [aidx], mask=mask)
        plsc.store_scatter(agg, [aidx], jnp.maximum(cur, vals), mask=mask)

    def detect(src_buf, g):
        src_v = src_buf[pl.ds(g * 16, 16)]
        _cnt, last = plsc.scan_count(src_v)
        return jnp.all(last)

    def rmw_window(src_buf, vals_buf):
        def group2(t, carry):
            g0 = t * 2
            g1 = g0 + 1
            sv0 = src_buf[pl.ds(g0 * 16, 16)]
            sv1 = src_buf[pl.ds(g1 * 16, 16)]
            _c0, last0 = plsc.scan_count(sv0)
            _c1, last1 = plsc.scan_count(sv1)
            nodup = jnp.logical_and(jnp.all(last0), jnp.all(last1))
            aidxs, valss = [], []
            for g, sv in ((g0, sv0), (g1, sv1)):
                erow = g * 16 + iota
                for f in range(8):
                    aidxs.append(f * NPAD + sv)
                    valss.append(plsc.load_gather(vals_buf,
                                                  [erow, jnp.full((16,), f,
                                                         jnp.int32)]))
            for half in range(2):
                curs = []
                for f in range(8):
                    curs.append(plsc.load_gather(agg, [aidxs[half * 8 + f]]))
                for f in range(8):
                    plsc.store_scatter(
                        agg, [aidxs[half * 8 + f]],
                        jnp.maximum(curs[f], valss[half * 8 + f]))

            @pl.when(jnp.logical_not(nodup))
            def _fixup():
                for g in (g0, g1):
                    for v in range(8):
                        rmw_pair(src_buf, vals_buf, g, v, lo_mask)
                    for v in range(8):
                        rmw_pair(src_buf, vals_buf, g, v, hi_mask)
            return carry
        lax.fori_loop(0, W_E // 32, group2, 0)

    srcs = (srcA, srcB)
    trgs = (trgA, trgB)
    gidxs = (gidxA, gidxB)
    valss = (valsA, valsB)
    sems = (semA, semB)
    semis = (semIA, semIB)

    issue_idx(0, srcA, trgA, semIA)
    wait_idx(0, srcA, trgA, semIA)
    fire_gather(trgA, gidxA, valsA, semA)
    issue_idx(1, srcB, trgB, semIB)

    def outer(i, _):
        for b in range(2):
            w = i * 2 + b
            nxt = 1 - b

            @pl.when(w + 1 < N_WINDOWS)
            def _fire():
                wait_idx(w + 1, srcs[nxt], trgs[nxt], semis[nxt])
                fire_gather(trgs[nxt], gidxs[nxt], valss[nxt], sems[nxt])
            pltpu.make_async_copy(h_hbm.at[gidxs[b]], valss[b], sems[b]).wait()
            rmw_window(srcs[b], valss[b])

            @pl.when(w + 2 < N_WINDOWS)
            def _issue():
                issue_idx(w + 2, srcs[b], trgs[b], semis[b])
        return _
    lax.fori_loop(0, N_WINDOWS // 2, outer, 0)

    pltpu.sync_copy(agg.at[pl.ds(0, FG * NPAD)], out_hbm.at[c, s])


def _segmax(h_r, src, trg):
    mesh = plsc.VectorSubcoreMesh(core_axis_name="c", subcore_axis_name="s")
    kfn = pl.kernel(
        _segmax_body,
        out_type=jax.ShapeDtypeStruct((NC, NS, FG * NPAD), jnp.float32),
        mesh=mesh,
        compiler_params=pltpu.CompilerParams(use_tc_tiling_on_sc=False,
                                             needs_layout_passes=False),
        scratch_types=[
            pltpu.VMEM((W_E,), jnp.int32),
            pltpu.VMEM((W_E,), jnp.int32),
            pltpu.VMEM((W_E + 16,), jnp.int32),
            pltpu.VMEM((W_E + 16,), jnp.int32),
            pltpu.VMEM((W_E,), jnp.int32),
            pltpu.VMEM((W_E,), jnp.int32),
            pltpu.VMEM((W_E, FG), jnp.float32),
            pltpu.VMEM((W_E, FG), jnp.float32),
            pltpu.VMEM((FG * NPAD + 16,), jnp.float32),
            pltpu.SemaphoreType.DMA,
            pltpu.SemaphoreType.DMA,
            pltpu.SemaphoreType.DMA,
            pltpu.SemaphoreType.DMA,
        ],
    )
    return kfn(h_r, src, trg)


def _final_body(x_ref, a_ref, w_ref, out_ref):
    m = jnp.maximum(a_ref[0], a_ref[1])
    out_ref[...] = (
        jnp.dot(x_ref[...], w_ref[:D, :], preferred_element_type=jnp.float32)
        + lax.dot_general(m, w_ref[D:, :], (((0,), (0,)), ((), ())),
                          preferred_element_type=jnp.float32)
    )


def _final(x, aggT, W):
    return pl.pallas_call(
        _final_body,
        out_shape=jax.ShapeDtypeStruct((N, D), jnp.float32),
    )(x, aggT, W)


@jax.jit
def _run(x, idx, fc_W, fc_b, W):
    h = _compute_h(x, fc_W, fc_b)
    h_r = h.reshape(N * NS, FG)
    aggT = _segmax(h_r, idx[0], idx[1])
    aggT = aggT.reshape(NC, NS, FG, NPAD)[..., :N].reshape(NC, D, N)
    return _final(x, aggT, W)


def kernel(input_matrix, adjacency_coo_matrix, fc_W, fc_b, W):
    return _run(input_matrix, adjacency_coo_matrix, fc_W, fc_b, W)
